# baseline (device time: 237240 ns/iter reference)
import jax
import jax.numpy as jnp
from jax import lax
from jax.experimental import pallas as pl
from jax.experimental.pallas import tpu as pltpu

N_DEV = 16
B, SQ, SKV = 2, 512, 512
H_PER = 8
DH = 64
D_MODEL = 768
CHUNK = B * SQ // N_DEV
CPB = SQ // CHUNK
WINDOW = 128
HOPS = N_DEV - 1


def kernel(x, Wq, K_ext, V_ext, Wo):

    def body(x_ref, wq_ref, k_hbm, v_hbm, wo_ref, out_ref,
             k_vmem, v_vmem, rs_recv, ag_recv, copy_sems,
             rs_send_sems, rs_recv_sems, ag_send_sems, ag_recv_sems):
        my = lax.axis_index("i")
        left = lax.rem(my + N_DEV - 1, N_DEV)
        right = lax.rem(my + 1, N_DEV)

        kcp = pltpu.make_async_copy(
            k_hbm.at[:, :, pl.ds(my * H_PER, H_PER), :], k_vmem,
            copy_sems.at[0])
        vcp = pltpu.make_async_copy(
            v_hbm.at[:, :, pl.ds(my * H_PER, H_PER), :], v_vmem,
            copy_sems.at[1])
        kcp.start()
        vcp.start()

        barrier = pltpu.get_barrier_semaphore()
        for nbr in (left, right):
            pl.semaphore_signal(
                barrier, inc=1,
                device_id=(nbr,), device_id_type=pl.DeviceIdType.MESH,
            )
        pl.semaphore_wait(barrier, 2)

        qi = lax.broadcasted_iota(jnp.int32, (SQ, SKV), 0)
        ki = lax.broadcasted_iota(jnp.int32, (SQ, SKV), 1)
        mask = jnp.abs(qi - ki) <= WINDOW
        q = [jnp.dot(x_ref[b], wq_ref[:, :],
                     preferred_element_type=jnp.float32) for b in range(B)]
        kcp.wait()
        vcp.wait()
        for b in range(B):
            ctx_cols = []
            for h in range(H_PER):
                qh = q[b][:, h * DH:(h + 1) * DH]
                kh = k_vmem[b, :, h, :]
                vh = v_vmem[b, :, h, :]
                s = lax.dot_general(
                    qh, kh, (((1,), (1,)), ((), ())),
                    preferred_element_type=jnp.float32) * 0.125
                s = jnp.where(mask, s, -1e9)
                s = s - jnp.max(s, axis=-1, keepdims=True)
                w = jnp.exp(s)
                w = w / jnp.sum(w, axis=-1, keepdims=True)
                ctx_cols.append(jnp.dot(w, vh,
                                        preferred_element_type=jnp.float32))
            ctx_b = jnp.concatenate(ctx_cols, axis=1)
            out_ref[b, :, :] = jnp.dot(
                ctx_b, wo_ref[:, :], preferred_element_type=jnp.float32)

        def chunk_slice(c):
            return (c // CPB, pl.ds(lax.rem(c, CPB) * CHUNK, CHUNK))

        for s in range(HOPS):
            cs = chunk_slice(lax.rem(my - s + 2 * N_DEV, N_DEV))
            cr = chunk_slice(lax.rem(my - s - 1 + 2 * N_DEV, N_DEV))
            rdma = pltpu.make_async_remote_copy(
                src_ref=out_ref.at[cs[0], cs[1], :],
                dst_ref=rs_recv.at[s],
                send_sem=rs_send_sems.at[s],
                recv_sem=rs_recv_sems.at[s],
                device_id=(right,),
                device_id_type=pl.DeviceIdType.MESH,
            )
            rdma.start()
            rdma.wait()
            out_ref[cr[0], cr[1], :] = out_ref[cr[0], cr[1], :] + rs_recv[s]

        for s in range(HOPS):
            cs = chunk_slice(lax.rem(my + 1 - s + 2 * N_DEV, N_DEV))
            cg = chunk_slice(lax.rem(my - s + 2 * N_DEV, N_DEV))
            rdma = pltpu.make_async_remote_copy(
                src_ref=out_ref.at[cs[0], cs[1], :],
                dst_ref=ag_recv.at[s],
                send_sem=ag_send_sems.at[s],
                recv_sem=ag_recv_sems.at[s],
                device_id=(right,),
                device_id_type=pl.DeviceIdType.MESH,
            )
            rdma.start()
            rdma.wait()
            out_ref[cg[0], cg[1], :] = ag_recv[s]

    vmem = pl.BlockSpec(memory_space=pltpu.MemorySpace.VMEM)
    hbm = pl.BlockSpec(memory_space=pltpu.MemorySpace.HBM)
    return pl.pallas_call(
        body,
        out_shape=jax.ShapeDtypeStruct((B, SQ, D_MODEL), jnp.float32),
        in_specs=[vmem, vmem, hbm, hbm, vmem],
        out_specs=vmem,
        scratch_shapes=[
            pltpu.VMEM((B, SQ, H_PER, DH), jnp.float32),
            pltpu.VMEM((B, SQ, H_PER, DH), jnp.float32),
            pltpu.VMEM((HOPS, CHUNK, D_MODEL), jnp.float32),
            pltpu.VMEM((HOPS, CHUNK, D_MODEL), jnp.float32),
            pltpu.SemaphoreType.DMA((2,)),
            pltpu.SemaphoreType.DMA((HOPS,)),
            pltpu.SemaphoreType.DMA((HOPS,)),
            pltpu.SemaphoreType.DMA((HOPS,)),
            pltpu.SemaphoreType.DMA((HOPS,)),
        ],
        compiler_params=pltpu.CompilerParams(collective_id=0),
    )(x, Wq, K_ext, V_ext, Wo)


# device time: 230366 ns/iter; 1.0298x vs baseline; 1.0298x over previous
import jax
import jax.numpy as jnp
from jax import lax
from jax.experimental import pallas as pl
from jax.experimental.pallas import tpu as pltpu

N_DEV = 16
B, SQ, SKV = 2, 512, 512
H_PER = 8
DH = 64
D_MODEL = 768
HALF = D_MODEL // 2
CHUNK = B * SQ // N_DEV
CPB = SQ // CHUNK
WINDOW = 128
HOPS = N_DEV - 1


def kernel(x, Wq, K_ext, V_ext, Wo):
    i = lax.axis_index("i")
    K = lax.dynamic_slice_in_dim(K_ext, i * H_PER, H_PER, axis=2)
    V = lax.dynamic_slice_in_dim(V_ext, i * H_PER, H_PER, axis=2)

    def body(x_ref, wq_ref, k_ref, v_ref, wo_ref, out_ref,
             rs_recv_r, rs_recv_l, ag_recv_r, ag_recv_l,
             rs_send_sems_r, rs_recv_sems_r, rs_send_sems_l, rs_recv_sems_l,
             ag_send_sems_r, ag_recv_sems_r, ag_send_sems_l, ag_recv_sems_l):
        my = lax.axis_index("i")
        left = lax.rem(my + N_DEV - 1, N_DEV)
        right = lax.rem(my + 1, N_DEV)

        barrier = pltpu.get_barrier_semaphore()
        for nbr in (left, right):
            pl.semaphore_signal(
                barrier, inc=1,
                device_id=(nbr,), device_id_type=pl.DeviceIdType.MESH,
            )
        pl.semaphore_wait(barrier, 2)

        qi = lax.broadcasted_iota(jnp.int32, (SQ, SKV), 0)
        ki = lax.broadcasted_iota(jnp.int32, (SQ, SKV), 1)
        mask = jnp.abs(qi - ki) <= WINDOW
        for b in range(B):
            q_b = jnp.dot(x_ref[b], wq_ref[:, :],
                          preferred_element_type=jnp.float32)
            ctx_cols = []
            for h in range(H_PER):
                qh = q_b[:, h * DH:(h + 1) * DH]
                kh = k_ref[b, :, h, :]
                vh = v_ref[b, :, h, :]
                s = lax.dot_general(
                    qh, kh, (((1,), (1,)), ((), ())),
                    preferred_element_type=jnp.float32) * 0.125
                s = jnp.where(mask, s, -1e9)
                s = s - jnp.max(s, axis=-1, keepdims=True)
                w = jnp.exp(s)
                w = w / jnp.sum(w, axis=-1, keepdims=True)
                ctx_cols.append(jnp.dot(w, vh,
                                        preferred_element_type=jnp.float32))
            ctx_b = jnp.concatenate(ctx_cols, axis=1)
            out_ref[b, :, :] = jnp.dot(
                ctx_b, wo_ref[:, :], preferred_element_type=jnp.float32)

        R = slice(0, HALF)
        L = slice(HALF, D_MODEL)

        def chunk(c):
            return c // CPB, pl.ds(lax.rem(c, CPB) * CHUNK, CHUNK)

        def send(src, dst, ssem, rsem, dev):
            rdma = pltpu.make_async_remote_copy(
                src_ref=src, dst_ref=dst, send_sem=ssem, recv_sem=rsem,
                device_id=(dev,), device_id_type=pl.DeviceIdType.MESH)
            rdma.start()
            return rdma

        rs_rdmas = []
        for s in range(HOPS):
            bR, rR = chunk(lax.rem(my - s + 2 * N_DEV, N_DEV))
            bL, rL = chunk(lax.rem(my + s, N_DEV))
            rd_r = send(out_ref.at[bR, rR, R], rs_recv_r.at[s],
                        rs_send_sems_r.at[s], rs_recv_sems_r.at[s], right)
            rd_l = send(out_ref.at[bL, rL, L], rs_recv_l.at[s],
                        rs_send_sems_l.at[s], rs_recv_sems_l.at[s], left)
            rs_rdmas += [rd_r, rd_l]
            rd_r.wait_recv()
            rd_l.wait_recv()
            bR, rR = chunk(lax.rem(my - s - 1 + 2 * N_DEV, N_DEV))
            bL, rL = chunk(lax.rem(my + s + 1, N_DEV))
            out_ref[bR, rR, R] = out_ref[bR, rR, R] + rs_recv_r[s]
            out_ref[bL, rL, L] = out_ref[bL, rL, L] + rs_recv_l[s]
        for rd in rs_rdmas:
            rd.wait_send()

        ag_rdmas = []
        for s in range(HOPS):
            if s == 0:
                bR, rR = chunk(lax.rem(my + 1, N_DEV))
                bL, rL = chunk(lax.rem(my - 1 + N_DEV, N_DEV))
                src_r = out_ref.at[bR, rR, R]
                src_l = out_ref.at[bL, rL, L]
            else:
                src_r = ag_recv_r.at[s - 1]
                src_l = ag_recv_l.at[s - 1]
            rd_r = send(src_r, ag_recv_r.at[s],
                        ag_send_sems_r.at[s], ag_recv_sems_r.at[s], right)
            rd_l = send(src_l, ag_recv_l.at[s],
                        ag_send_sems_l.at[s], ag_recv_sems_l.at[s], left)
            ag_rdmas += [rd_r, rd_l]
            rd_r.wait_recv()
            rd_l.wait_recv()
        for s in range(HOPS):
            bR, rR = chunk(lax.rem(my - s + 2 * N_DEV, N_DEV))
            bL, rL = chunk(lax.rem(my + s, N_DEV))
            out_ref[bR, rR, R] = ag_recv_r[s]
            out_ref[bL, rL, L] = ag_recv_l[s]
        for rd in ag_rdmas:
            rd.wait_send()

    vmem = pl.BlockSpec(memory_space=pltpu.MemorySpace.VMEM)
    return pl.pallas_call(
        body,
        out_shape=jax.ShapeDtypeStruct((B, SQ, D_MODEL), jnp.float32),
        in_specs=[vmem] * 5,
        out_specs=vmem,
        scratch_shapes=[
            pltpu.VMEM((HOPS, CHUNK, HALF), jnp.float32),
            pltpu.VMEM((HOPS, CHUNK, HALF), jnp.float32),
            pltpu.VMEM((HOPS, CHUNK, HALF), jnp.float32),
            pltpu.VMEM((HOPS, CHUNK, HALF), jnp.float32),
        ] + [pltpu.SemaphoreType.DMA((HOPS,))] * 8,
        compiler_params=pltpu.CompilerParams(collective_id=0),
    )(x, Wq, K, V, Wo)


# device time: 160408 ns/iter; 1.4790x vs baseline; 1.4361x over previous
import jax
import jax.numpy as jnp
from jax import lax
from jax.experimental import pallas as pl
from jax.experimental.pallas import tpu as pltpu

N_DEV = 16
B, SQ, SKV = 2, 512, 512
H_PER = 8
DH = 64
D_MODEL = 768
CHUNK = B * SQ // N_DEV
CPB = SQ // CHUNK
WINDOW = 128
HOPS = N_DEV - 1


def kernel(x, Wq, K_ext, V_ext, Wo):
    i = lax.axis_index("i")
    K = lax.dynamic_slice_in_dim(K_ext, i * H_PER, H_PER, axis=2)
    V = lax.dynamic_slice_in_dim(V_ext, i * H_PER, H_PER, axis=2)
    K = K.astype(jnp.bfloat16)
    V = V.astype(jnp.bfloat16)

    def body(x_ref, wq_ref, k_ref, v_ref, wo_ref, out_ref,
             snd, rs_recv, ag_recv,
             rs_send_sems, rs_recv_sems, ag_send_sems, ag_recv_sems):
        my = lax.axis_index("i")
        left = lax.rem(my + N_DEV - 1, N_DEV)
        right = lax.rem(my + 1, N_DEV)

        barrier = pltpu.get_barrier_semaphore()
        for nbr in (left, right):
            pl.semaphore_signal(
                barrier, inc=1,
                device_id=(nbr,), device_id_type=pl.DeviceIdType.MESH,
            )
        pl.semaphore_wait(barrier, 2)

        qi = lax.broadcasted_iota(jnp.int32, (SQ, SKV), 0)
        ki = lax.broadcasted_iota(jnp.int32, (SQ, SKV), 1)
        mask = jnp.abs(qi - ki) <= WINDOW
        for b in range(B):
            q_b = jnp.dot(x_ref[b], wq_ref[:, :],
                          preferred_element_type=jnp.float32)
            q_b = q_b.astype(jnp.bfloat16)
            ctx_cols = []
            for h in range(H_PER):
                qh = q_b[:, h * DH:(h + 1) * DH]
                kh = k_ref[b, :, h, :]
                vh = v_ref[b, :, h, :]
                s = lax.dot_general(
                    qh, kh, (((1,), (1,)), ((), ())),
                    preferred_element_type=jnp.float32) * 0.125
                s = jnp.where(mask, s, -1e9)
                s = s - jnp.max(s, axis=-1, keepdims=True)
                w = jnp.exp(s)
                w = (w / jnp.sum(w, axis=-1, keepdims=True)).astype(jnp.bfloat16)
                ctx_cols.append(jnp.dot(w, vh,
                                        preferred_element_type=jnp.float32))
            ctx_b = jnp.concatenate(ctx_cols, axis=1)
            out_ref[b, :, :] = jnp.dot(
                ctx_b, wo_ref[:, :], preferred_element_type=jnp.float32)

        def chunk(c):
            return c // CPB, pl.ds(lax.rem(c, CPB) * CHUNK, CHUNK)

        def send(src, dst, ssem, rsem, dev):
            rdma = pltpu.make_async_remote_copy(
                src_ref=src, dst_ref=dst, send_sem=ssem, recv_sem=rsem,
                device_id=(dev,), device_id_type=pl.DeviceIdType.MESH)
            rdma.start()
            return rdma

        rs_rdmas = []
        for s in range(HOPS):
            bc, rc = chunk(lax.rem(my - s + 2 * N_DEV, N_DEV))
            snd[s] = out_ref[bc, rc, :].astype(jnp.bfloat16)
            rd = send(snd.at[s], rs_recv.at[s],
                      rs_send_sems.at[s], rs_recv_sems.at[s], right)
            rs_rdmas.append(rd)
            rd.wait_recv()
            bc, rc = chunk(lax.rem(my - s - 1 + 2 * N_DEV, N_DEV))
            out_ref[bc, rc, :] = (out_ref[bc, rc, :]
                                  + rs_recv[s].astype(jnp.float32))
        for rd in rs_rdmas:
            rd.wait_send()

        ag_rdmas = []
        for s in range(HOPS):
            if s == 0:
                bc, rc = chunk(lax.rem(my + 1, N_DEV))
                snd[HOPS] = out_ref[bc, rc, :].astype(jnp.bfloat16)
                src = snd.at[HOPS]
            else:
                src = ag_recv.at[s - 1]
            rd = send(src, ag_recv.at[s],
                      ag_send_sems.at[s], ag_recv_sems.at[s], right)
            ag_rdmas.append(rd)
            rd.wait_recv()
        for s in range(HOPS):
            bc, rc = chunk(lax.rem(my - s + 2 * N_DEV, N_DEV))
            out_ref[bc, rc, :] = ag_recv[s].astype(jnp.float32)
        for rd in ag_rdmas:
            rd.wait_send()

    vmem = pl.BlockSpec(memory_space=pltpu.MemorySpace.VMEM)
    return pl.pallas_call(
        body,
        out_shape=jax.ShapeDtypeStruct((B, SQ, D_MODEL), jnp.float32),
        in_specs=[vmem] * 5,
        out_specs=vmem,
        scratch_shapes=[
            pltpu.VMEM((HOPS + 1, CHUNK, D_MODEL), jnp.bfloat16),
            pltpu.VMEM((HOPS, CHUNK, D_MODEL), jnp.bfloat16),
            pltpu.VMEM((HOPS, CHUNK, D_MODEL), jnp.bfloat16),
        ] + [pltpu.SemaphoreType.DMA((HOPS,))] * 4,
        compiler_params=pltpu.CompilerParams(collective_id=0),
    )(x, Wq, K, V, Wo)
